# baseline (speedup 1.0000x reference)
import functools

import jax
import jax.numpy as jnp
import numpy as np
from jax import lax
from jax.experimental import pallas as pl
from jax.experimental.pallas import tpu as pltpu
from jax.experimental.pallas import tpu_sc as plsc

_ATOM = {'C': 1, 'CA': 2, 'CB': 3, 'CD': 4, 'CD1': 5, 'CD2': 6, 'CE': 7, 'CE1': 8, 'CE2': 9, 'CE3': 10, 'CG': 11, 'CG1': 12, 'CG2': 13, 'CH2': 14, 'CZ': 15, 'CZ2': 16, 'CZ3': 17, 'H': 18, 'HA': 19, 'HB': 20, 'HD': 21, 'HD1': 22, 'HD2': 23, 'HE': 24, 'HE1': 25, 'HE2': 26, 'HE3': 27, 'HG': 28, 'HG1': 29, 'HG2': 30, 'HH': 31, 'HH1': 32, 'HH2': 33, 'HZ': 34, 'HZ2': 35, 'HZ3': 36, 'N': 37, 'ND1': 38, 'ND2': 39, 'NE': 40, 'NE1': 41, 'NE2': 42, 'NH1': 43, 'NH2': 44, 'NZ': 45, 'O': 46, 'OD': 47, 'OE': 48, 'OG': 49, 'OG1': 50, 'OH': 51, 'OXT': 52, 'SD': 53, 'SG': 54}
_AA_N = 20
_BACKBONE = ['N', 'CA', 'C', 'O']
_SIDECHAIN = {'ALA': ['CB'], 'ARG': ['CB', 'CG', 'CD', 'NE', 'CZ', 'NH1', 'NH2'], 'CYS': ['CB', 'SG'], 'GLY': [], 'HIS': ['CB', 'CG', 'ND1', 'CD2', 'CE1', 'NE2'], 'ILE': ['CB', 'CG1', 'CG2', 'CD1'], 'LEU': ['CB', 'CG', 'CD1', 'CD2'], 'LYS': ['CB', 'CG', 'CD', 'CE', 'NZ'], 'MET': ['CB', 'CG', 'SD', 'CE'], 'PHE': ['CB', 'CG', 'CD1', 'CD2', 'CE1', 'CE2', 'CZ'], 'PRO': ['CB', 'CG', 'CD'], 'SER': ['CB', 'OG'], 'THR': ['CB', 'OG1', 'CG2'], 'TRP': ['CB', 'CG', 'CD1', 'NE1', 'CD2', 'CE2', 'CE3', 'CZ2', 'CZ3', 'CH2'], 'TYR': ['CB', 'CG', 'CD1', 'CD2', 'CE1', 'CE2', 'CZ', 'OH'], 'VAL': ['CB', 'CG1', 'CG2']}
_AA_NAMES = ['ALA', 'ARG', 'ASN', 'ASP', 'CYS', 'GLN', 'GLU', 'GLY', 'HIS', 'ILE', 'LEU', 'LYS', 'MET', 'PHE', 'PRO', 'SER', 'THR', 'TRP', 'TYR', 'VAL']
_MAX_HEAVY = 14
_N_CODE = 37

_LUT_NP = np.full((_AA_N + 1) * 64, _MAX_HEAVY, dtype=np.int32)
for _i, _name in enumerate(_AA_NAMES):
    _aa_code = _i + 1
    for _r, _atom_name in enumerate(_BACKBONE + _SIDECHAIN.get(_name, [])):
        _LUT_NP[_aa_code * 64 + _ATOM[_atom_name]] = _r

_LUT_PACKED = []
for _j in range(176):
    _w = 0
    for _t in range(8):
        _e = int(_LUT_NP[8 * _j + _t]) if 8 * _j + _t < _LUT_NP.size else _MAX_HEAVY
        _w |= _e << (4 * _t)
    _LUT_PACKED.append(_w - (1 << 32) if _w >= (1 << 31) else _w)

B, L, D = 8, 2048, 256
NC, NS = 2, 16
CHUNK = L // 4
PAD = 32
ROWBUF = L + 2 * PAD
EXTN = 560
DCHUNK = 64
NDMA = CHUNK // DCHUNK
NBUF = 4


def _body(x_hbm, cb_hbm, out_hbm,
          cbbuf, lutbuf, sbuf, vbuf, ckbuf, destbuf, xbuf,
          isem0, isem1, isem2, isem3, osem0, osem1, osem2, osem3):
    isems = [isem0, isem1, isem2, isem3]
    osems = [osem0, osem1, osem2, osem3]
    wid = lax.axis_index("s") * NC + lax.axis_index("c")
    b = wid // 4
    base = (wid % 4) * CHUNK
    row0 = b * L

    zeros = jnp.zeros((16,), jnp.int32)
    cbbuf[pl.ds(0, 16)] = zeros
    cbbuf[pl.ds(16, 16)] = zeros
    cbbuf[pl.ds(PAD + L, 16)] = zeros
    cbbuf[pl.ds(PAD + L + 16, 16)] = zeros
    pltpu.sync_copy(cb_hbm.at[pl.ds(B * L, len(_LUT_PACKED))], lutbuf)
    pltpu.sync_copy(cb_hbm.at[pl.ds(row0, L)], cbbuf.at[pl.ds(PAD, L)])

    loads = {}
    for d in range(NBUF):
        g = base + d * DCHUNK
        loads[d] = pltpu.async_copy(
            x_hbm.at[b, pl.ds(g, DCHUNK)], xbuf.at[d], isems[d])

    lane = jax.lax.iota(jnp.int32, 16)

    def pass_a(t, _):
        o = 16 * t
        p = base - 32 + o
        idx = p + lane
        ty = cbbuf[pl.ds(PAD + p, 16)] & 63
        sbuf[pl.ds(o, 16)] = jnp.where(ty == _N_CODE, idx, -1)
        vbuf[pl.ds(o, 16)] = ty * 2048 + idx
        return _

    lax.fori_loop(0, EXTN // 16, pass_a, None)

    for k in (1, 2, 4, 8):
        def pass_b(t, _, k=k):
            o = 16 + 16 * t
            s = jnp.maximum(sbuf[pl.ds(o, 16)], sbuf[pl.ds(o - k, 16)])
            sbuf[pl.ds(o, 16)] = s
            return _

        lax.fori_loop(0, (EXTN - 16) // 16, pass_b, None)

    def pass_c(t, _):
        o = 16 + 16 * t
        p = base - 32 + o
        idx = p + lane
        cb = cbbuf[pl.ds(PAD + p, 16)]
        ty = cb & 63
        seg = sbuf[pl.ds(o, 16)]
        th = (idx - seg).astype(jnp.uint32)
        vi1 = vbuf[pl.ds(o, 16)] - 1
        dup = jnp.zeros((16,), jnp.bool_)
        for k in range(1, 16):
            vk = vbuf[pl.ds(o - k, 16)]
            dup = dup | ((vi1 - vk).astype(jnp.uint32) < th)
        cb_seg = plsc.load_gather(cbbuf, [seg + PAD])
        lk = cb_seg - (cb_seg & 63) + ty
        word = plsc.load_gather(lutbuf, [lk >> 3])
        hr = (word >> ((lk & 7) * 4)) & 15
        keyl = jnp.where((~dup) & (hr < _MAX_HEAVY), hr,
                         _MAX_HEAVY + idx - seg)
        segm = jnp.where((idx >= 0) & (idx < L), seg, -999)
        ckbuf[pl.ds(o, 16)] = segm * 4096 + keyl
        return _

    lax.fori_loop(0, (EXTN - 16) // 16, pass_c, None)

    def pass_d(t, _):
        o = 32 + 16 * t
        cki = ckbuf[pl.ds(o, 16)]
        key = (cki & 4095).astype(jnp.uint32)
        cki1 = cki - 1
        cnt = jnp.zeros((16,), jnp.int32)
        one = jnp.ones((16,), jnp.int32)
        for k in range(-15, 16):
            if k == 0:
                continue
            ckj = ckbuf[pl.ds(o + k, 16)]
            cond = (cki1 - ckj).astype(jnp.uint32) < key
            cnt = cnt + jnp.where(cond, one, 0)
        destbuf[t // 4, pl.ds((t % 4) * 16, 16)] = row0 + (cki >> 12) + cnt
        return _

    lax.fori_loop(0, CHUNK // 16, pass_d, None)

    scats = {}
    for d in range(NDMA):
        bi = d % NBUF
        if d >= NBUF:
            scats[d - NBUF].wait()
            g = base + d * DCHUNK
            loads[d] = pltpu.async_copy(
                x_hbm.at[b, pl.ds(g, DCHUNK)], xbuf.at[bi], isems[bi])
        loads[d].wait()
        scats[d] = pltpu.async_copy(
            xbuf.at[bi], out_hbm.at[destbuf.at[d]], osems[bi])

    for d in range(NDMA - NBUF, NDMA):
        scats[d].wait()


_mesh = plsc.VectorSubcoreMesh(core_axis_name="c", subcore_axis_name="s")

_sc_permute = functools.partial(
    pl.kernel,
    out_type=jax.ShapeDtypeStruct((B * L, D), jnp.float32),
    mesh=_mesh,
    compiler_params=pltpu.CompilerParams(needs_layout_passes=False),
    scratch_types=[
        pltpu.VMEM((ROWBUF,), jnp.int32),
        pltpu.VMEM((len(_LUT_PACKED),), jnp.int32),
        pltpu.VMEM((EXTN,), jnp.int32),
        pltpu.VMEM((EXTN,), jnp.int32),
        pltpu.VMEM((EXTN,), jnp.int32),
        pltpu.VMEM((NDMA, DCHUNK), jnp.int32),
        pltpu.VMEM((NBUF, DCHUNK, D), jnp.float32),
        pltpu.SemaphoreType.DMA,
        pltpu.SemaphoreType.DMA,
        pltpu.SemaphoreType.DMA,
        pltpu.SemaphoreType.DMA,
        pltpu.SemaphoreType.DMA,
        pltpu.SemaphoreType.DMA,
        pltpu.SemaphoreType.DMA,
        pltpu.SemaphoreType.DMA,
    ],
)(_body)


@jax.jit
def kernel(x, atom_type, aa_type):
    comb = (aa_type.astype(jnp.int32) * 64
            + atom_type.astype(jnp.int32)).reshape(B * L)
    comb = jnp.concatenate(
        [comb, jnp.asarray(_LUT_PACKED, dtype=jnp.int32)])
    out = _sc_permute(x, comb)
    return out.reshape(B, L, D)


# speedup vs baseline: 1.0658x; 1.0658x over previous
import functools

import jax
import jax.numpy as jnp
import numpy as np
from jax import lax
from jax.experimental import pallas as pl
from jax.experimental.pallas import tpu as pltpu
from jax.experimental.pallas import tpu_sc as plsc

_ATOM = {'C': 1, 'CA': 2, 'CB': 3, 'CD': 4, 'CD1': 5, 'CD2': 6, 'CE': 7, 'CE1': 8, 'CE2': 9, 'CE3': 10, 'CG': 11, 'CG1': 12, 'CG2': 13, 'CH2': 14, 'CZ': 15, 'CZ2': 16, 'CZ3': 17, 'H': 18, 'HA': 19, 'HB': 20, 'HD': 21, 'HD1': 22, 'HD2': 23, 'HE': 24, 'HE1': 25, 'HE2': 26, 'HE3': 27, 'HG': 28, 'HG1': 29, 'HG2': 30, 'HH': 31, 'HH1': 32, 'HH2': 33, 'HZ': 34, 'HZ2': 35, 'HZ3': 36, 'N': 37, 'ND1': 38, 'ND2': 39, 'NE': 40, 'NE1': 41, 'NE2': 42, 'NH1': 43, 'NH2': 44, 'NZ': 45, 'O': 46, 'OD': 47, 'OE': 48, 'OG': 49, 'OG1': 50, 'OH': 51, 'OXT': 52, 'SD': 53, 'SG': 54}
_AA_N = 20
_BACKBONE = ['N', 'CA', 'C', 'O']
_SIDECHAIN = {'ALA': ['CB'], 'ARG': ['CB', 'CG', 'CD', 'NE', 'CZ', 'NH1', 'NH2'], 'CYS': ['CB', 'SG'], 'GLY': [], 'HIS': ['CB', 'CG', 'ND1', 'CD2', 'CE1', 'NE2'], 'ILE': ['CB', 'CG1', 'CG2', 'CD1'], 'LEU': ['CB', 'CG', 'CD1', 'CD2'], 'LYS': ['CB', 'CG', 'CD', 'CE', 'NZ'], 'MET': ['CB', 'CG', 'SD', 'CE'], 'PHE': ['CB', 'CG', 'CD1', 'CD2', 'CE1', 'CE2', 'CZ'], 'PRO': ['CB', 'CG', 'CD'], 'SER': ['CB', 'OG'], 'THR': ['CB', 'OG1', 'CG2'], 'TRP': ['CB', 'CG', 'CD1', 'NE1', 'CD2', 'CE2', 'CE3', 'CZ2', 'CZ3', 'CH2'], 'TYR': ['CB', 'CG', 'CD1', 'CD2', 'CE1', 'CE2', 'CZ', 'OH'], 'VAL': ['CB', 'CG1', 'CG2']}
_AA_NAMES = ['ALA', 'ARG', 'ASN', 'ASP', 'CYS', 'GLN', 'GLU', 'GLY', 'HIS', 'ILE', 'LEU', 'LYS', 'MET', 'PHE', 'PRO', 'SER', 'THR', 'TRP', 'TYR', 'VAL']
_MAX_HEAVY = 14
_N_CODE = 37

_LUT_NP = np.full((_AA_N + 1) * 64, _MAX_HEAVY, dtype=np.int32)
for _i, _name in enumerate(_AA_NAMES):
    _aa_code = _i + 1
    for _r, _atom_name in enumerate(_BACKBONE + _SIDECHAIN.get(_name, [])):
        _LUT_NP[_aa_code * 64 + _ATOM[_atom_name]] = _r

_LUT_PACKED = []
for _j in range(176):
    _w = 0
    for _t in range(8):
        _e = int(_LUT_NP[8 * _j + _t]) if 8 * _j + _t < _LUT_NP.size else _MAX_HEAVY
        _w |= _e << (4 * _t)
    _LUT_PACKED.append(_w - (1 << 32) if _w >= (1 << 31) else _w)

B, L, D = 8, 2048, 256
NC, NS = 2, 16
CHUNK = L // 4
PAD = 32
ROWBUF = L + 2 * PAD
EXTN = 560
DCHUNK = 128
NDMA = CHUNK // DCHUNK
NBUF = 3


def _body(x_hbm, cb_hbm, out_hbm,
          cbbuf, lutbuf, sbuf, vbuf, ckbuf, destbuf, xbuf,
          isem0, isem1, isem2, osem0, osem1, osem2):
    isems = [isem0, isem1, isem2]
    osems = [osem0, osem1, osem2]
    wid = lax.axis_index("s") * NC + lax.axis_index("c")
    b = wid // 4
    base = (wid % 4) * CHUNK
    row0 = b * L

    zeros = jnp.zeros((16,), jnp.int32)
    cbbuf[pl.ds(0, 16)] = zeros
    cbbuf[pl.ds(16, 16)] = zeros
    cbbuf[pl.ds(PAD + L, 16)] = zeros
    cbbuf[pl.ds(PAD + L + 16, 16)] = zeros
    lane0 = jax.lax.iota(jnp.int32, 16)
    for i in range(len(_LUT_PACKED) // 16):
        acc = jnp.zeros((16,), jnp.int32)
        for j in range(16):
            acc = jnp.where(lane0 == j, _LUT_PACKED[16 * i + j], acc)
        lutbuf[pl.ds(16 * i, 16)] = acc

    pltpu.sync_copy(cb_hbm.at[pl.ds(row0, L)], cbbuf.at[pl.ds(PAD, L)])

    loads = {}
    for d in range(NBUF):
        g = base + d * DCHUNK
        loads[d] = pltpu.async_copy(
            x_hbm.at[b, pl.ds(g, DCHUNK)], xbuf.at[d], isems[d])

    lane = jax.lax.iota(jnp.int32, 16)

    def pass_a(t, _):
        o = 16 * t
        p = base - 32 + o
        idx = p + lane
        ty = cbbuf[pl.ds(PAD + p, 16)] & 63
        sbuf[pl.ds(o, 16)] = jnp.where(ty == _N_CODE, idx, -1)
        vbuf[pl.ds(o, 16)] = ty * 2048 + idx
        return _

    lax.fori_loop(0, EXTN // 16, pass_a, None)

    for k in (1, 2, 4, 8):
        def pass_b(t, _, k=k):
            o = 16 + 16 * t
            s = jnp.maximum(sbuf[pl.ds(o, 16)], sbuf[pl.ds(o - k, 16)])
            sbuf[pl.ds(o, 16)] = s
            return _

        lax.fori_loop(0, (EXTN - 16) // 16, pass_b, None)

    def pass_c(t, _):
        o = 16 + 16 * t
        p = base - 32 + o
        idx = p + lane
        cb = cbbuf[pl.ds(PAD + p, 16)]
        ty = cb & 63
        seg = sbuf[pl.ds(o, 16)]
        th = (idx - seg).astype(jnp.uint32)
        vi1 = vbuf[pl.ds(o, 16)] - 1
        dup = jnp.zeros((16,), jnp.bool_)
        for k in range(1, 16):
            vk = vbuf[pl.ds(o - k, 16)]
            dup = dup | ((vi1 - vk).astype(jnp.uint32) < th)
        cb_seg = plsc.load_gather(cbbuf, [seg + PAD])
        lk = cb_seg - (cb_seg & 63) + ty
        word = plsc.load_gather(lutbuf, [lk >> 3])
        hr = (word >> ((lk & 7) * 4)) & 15
        keyl = jnp.where((~dup) & (hr < _MAX_HEAVY), hr,
                         _MAX_HEAVY + idx - seg)
        segm = jnp.where((idx >= 0) & (idx < L), seg, -999)
        ckbuf[pl.ds(o, 16)] = segm * 4096 + keyl
        return _

    lax.fori_loop(0, (EXTN - 16) // 16, pass_c, None)

    def pass_d(t, _):
        o = 32 + 16 * t
        cki = ckbuf[pl.ds(o, 16)]
        key = (cki & 4095).astype(jnp.uint32)
        cki1 = cki - 1
        cnt = jnp.zeros((16,), jnp.int32)
        one = jnp.ones((16,), jnp.int32)
        for k in range(-15, 16):
            if k == 0:
                continue
            ckj = ckbuf[pl.ds(o + k, 16)]
            cond = (cki1 - ckj).astype(jnp.uint32) < key
            cnt = cnt + jnp.where(cond, one, 0)
        vpc = DCHUNK // 16
        destbuf[t // vpc, pl.ds((t % vpc) * 16, 16)] = row0 + (cki >> 12) + cnt
        return _

    lax.fori_loop(0, CHUNK // 16, pass_d, None)

    scats = {}
    for d in range(NDMA):
        bi = d % NBUF
        if d >= NBUF:
            scats[d - NBUF].wait()
            g = base + d * DCHUNK
            loads[d] = pltpu.async_copy(
                x_hbm.at[b, pl.ds(g, DCHUNK)], xbuf.at[bi], isems[bi])
        loads[d].wait()
        scats[d] = pltpu.async_copy(
            xbuf.at[bi], out_hbm.at[destbuf.at[d]], osems[bi])

    for d in range(NDMA - NBUF, NDMA):
        scats[d].wait()


_mesh = plsc.VectorSubcoreMesh(core_axis_name="c", subcore_axis_name="s")

_sc_permute = functools.partial(
    pl.kernel,
    out_type=jax.ShapeDtypeStruct((B * L, D), jnp.float32),
    mesh=_mesh,
    compiler_params=pltpu.CompilerParams(needs_layout_passes=False),
    scratch_types=[
        pltpu.VMEM((ROWBUF,), jnp.int32),
        pltpu.VMEM((len(_LUT_PACKED),), jnp.int32),
        pltpu.VMEM((EXTN,), jnp.int32),
        pltpu.VMEM((EXTN,), jnp.int32),
        pltpu.VMEM((EXTN,), jnp.int32),
        pltpu.VMEM((NDMA, DCHUNK), jnp.int32),
        pltpu.VMEM((NBUF, DCHUNK, D), jnp.float32),
        pltpu.SemaphoreType.DMA,
        pltpu.SemaphoreType.DMA,
        pltpu.SemaphoreType.DMA,
        pltpu.SemaphoreType.DMA,
        pltpu.SemaphoreType.DMA,
        pltpu.SemaphoreType.DMA,
    ],
)(_body)


@jax.jit
def kernel(x, atom_type, aa_type):
    comb = (aa_type.astype(jnp.int32) * 64
            + atom_type.astype(jnp.int32)).reshape(B * L)
    out = _sc_permute(x, comb)
    return out.reshape(B, L, D)


# speedup vs baseline: 1.0705x; 1.0044x over previous
import functools

import jax
import jax.numpy as jnp
import numpy as np
from jax import lax
from jax.experimental import pallas as pl
from jax.experimental.pallas import tpu as pltpu
from jax.experimental.pallas import tpu_sc as plsc

_ATOM = {'C': 1, 'CA': 2, 'CB': 3, 'CD': 4, 'CD1': 5, 'CD2': 6, 'CE': 7, 'CE1': 8, 'CE2': 9, 'CE3': 10, 'CG': 11, 'CG1': 12, 'CG2': 13, 'CH2': 14, 'CZ': 15, 'CZ2': 16, 'CZ3': 17, 'H': 18, 'HA': 19, 'HB': 20, 'HD': 21, 'HD1': 22, 'HD2': 23, 'HE': 24, 'HE1': 25, 'HE2': 26, 'HE3': 27, 'HG': 28, 'HG1': 29, 'HG2': 30, 'HH': 31, 'HH1': 32, 'HH2': 33, 'HZ': 34, 'HZ2': 35, 'HZ3': 36, 'N': 37, 'ND1': 38, 'ND2': 39, 'NE': 40, 'NE1': 41, 'NE2': 42, 'NH1': 43, 'NH2': 44, 'NZ': 45, 'O': 46, 'OD': 47, 'OE': 48, 'OG': 49, 'OG1': 50, 'OH': 51, 'OXT': 52, 'SD': 53, 'SG': 54}
_AA_N = 20
_BACKBONE = ['N', 'CA', 'C', 'O']
_SIDECHAIN = {'ALA': ['CB'], 'ARG': ['CB', 'CG', 'CD', 'NE', 'CZ', 'NH1', 'NH2'], 'CYS': ['CB', 'SG'], 'GLY': [], 'HIS': ['CB', 'CG', 'ND1', 'CD2', 'CE1', 'NE2'], 'ILE': ['CB', 'CG1', 'CG2', 'CD1'], 'LEU': ['CB', 'CG', 'CD1', 'CD2'], 'LYS': ['CB', 'CG', 'CD', 'CE', 'NZ'], 'MET': ['CB', 'CG', 'SD', 'CE'], 'PHE': ['CB', 'CG', 'CD1', 'CD2', 'CE1', 'CE2', 'CZ'], 'PRO': ['CB', 'CG', 'CD'], 'SER': ['CB', 'OG'], 'THR': ['CB', 'OG1', 'CG2'], 'TRP': ['CB', 'CG', 'CD1', 'NE1', 'CD2', 'CE2', 'CE3', 'CZ2', 'CZ3', 'CH2'], 'TYR': ['CB', 'CG', 'CD1', 'CD2', 'CE1', 'CE2', 'CZ', 'OH'], 'VAL': ['CB', 'CG1', 'CG2']}
_AA_NAMES = ['ALA', 'ARG', 'ASN', 'ASP', 'CYS', 'GLN', 'GLU', 'GLY', 'HIS', 'ILE', 'LEU', 'LYS', 'MET', 'PHE', 'PRO', 'SER', 'THR', 'TRP', 'TYR', 'VAL']
_MAX_HEAVY = 14
_N_CODE = 37

_LUT_NP = np.full((_AA_N + 1) * 64, _MAX_HEAVY, dtype=np.int32)
for _i, _name in enumerate(_AA_NAMES):
    _aa_code = _i + 1
    for _r, _atom_name in enumerate(_BACKBONE + _SIDECHAIN.get(_name, [])):
        _LUT_NP[_aa_code * 64 + _ATOM[_atom_name]] = _r

_LUT_PACKED = []
for _j in range(176):
    _w = 0
    for _t in range(8):
        _e = int(_LUT_NP[8 * _j + _t]) if 8 * _j + _t < _LUT_NP.size else _MAX_HEAVY
        _w |= _e << (4 * _t)
    _LUT_PACKED.append(_w - (1 << 32) if _w >= (1 << 31) else _w)

B, L, D = 8, 2048, 256
NC, NS = 2, 16
CHUNK = L // 4
PAD = 32
ROWBUF = L + 2 * PAD
EXTN = 576
DCHUNK = 128
NDMA = CHUNK // DCHUNK
NBUF = 3


def _body(x_hbm, cb_hbm, out_hbm,
          cbbuf, lutbuf, sbuf, vbuf, ckbuf, destbuf, xbuf,
          isem0, isem1, isem2, osem0, osem1, osem2):
    isems = [isem0, isem1, isem2]
    osems = [osem0, osem1, osem2]
    wid = lax.axis_index("s") * NC + lax.axis_index("c")
    b = wid // 4
    base = (wid % 4) * CHUNK
    row0 = b * L

    zeros = jnp.zeros((16,), jnp.int32)
    cbbuf[pl.ds(0, 16)] = zeros
    cbbuf[pl.ds(16, 16)] = zeros
    cbbuf[pl.ds(PAD + L, 16)] = zeros
    cbbuf[pl.ds(PAD + L + 16, 16)] = zeros
    lane0 = jax.lax.iota(jnp.int32, 16)
    for i in range(len(_LUT_PACKED) // 16):
        acc = jnp.zeros((16,), jnp.int32)
        for j in range(16):
            acc = jnp.where(lane0 == j, _LUT_PACKED[16 * i + j], acc)
        lutbuf[pl.ds(16 * i, 16)] = acc

    pltpu.sync_copy(cb_hbm.at[pl.ds(row0, L)], cbbuf.at[pl.ds(PAD, L)])

    loads = {}
    for d in range(NBUF):
        g = base + d * DCHUNK
        loads[d] = pltpu.async_copy(
            x_hbm.at[b, pl.ds(g, DCHUNK)], xbuf.at[d], isems[d])

    lane = jax.lax.iota(jnp.int32, 16)

    sbuf[pl.ds(0, 16)] = jnp.full((16,), -1, jnp.int32)
    vbuf[pl.ds(0, 16)] = jnp.zeros((16,), jnp.int32)

    def pass_abc(t, _):
        o = 16 * t
        p = base - 48 + o
        idx = p + lane
        cb = cbbuf[pl.ds(PAD + p, 16)]
        ty = cb & 63
        vbuf[pl.ds(o, 16)] = ty * 2048 + idx
        s = jnp.where(ty == _N_CODE, idx, -1)
        for k in (1, 2, 4, 8):
            sbuf[pl.ds(o, 16)] = s
            s = jnp.maximum(s, sbuf[pl.ds(o - k, 16)])
        seg = s
        sbuf[pl.ds(o, 16)] = seg
        th = (idx - seg).astype(jnp.uint32)
        vi1 = ty * 2048 + idx - 1
        dup = jnp.zeros((16,), jnp.bool_)
        for k in range(1, 16):
            vk = vbuf[pl.ds(o - k, 16)]
            dup = dup | ((vi1 - vk).astype(jnp.uint32) < th)
        cb_seg = plsc.load_gather(cbbuf, [seg + PAD])
        lk = cb_seg - (cb_seg & 63) + ty
        word = plsc.load_gather(lutbuf, [lk >> 3])
        hr = (word >> ((lk & 7) * 4)) & 15
        keyl = jnp.where((~dup) & (hr < _MAX_HEAVY), hr,
                         _MAX_HEAVY + idx - seg)
        segm = jnp.where((idx >= 0) & (idx < L), seg, -999)
        ckbuf[pl.ds(o, 16)] = segm * 4096 + keyl
        return _

    lax.fori_loop(1, EXTN // 16, pass_abc, None)

    def pass_d(t, _):
        o = 48 + 16 * t
        cki = ckbuf[pl.ds(o, 16)]
        key = (cki & 4095).astype(jnp.uint32)
        cki1 = cki - 1
        cnt = jnp.zeros((16,), jnp.int32)
        one = jnp.ones((16,), jnp.int32)
        for k in range(-15, 16):
            if k == 0:
                continue
            ckj = ckbuf[pl.ds(o + k, 16)]
            cond = (cki1 - ckj).astype(jnp.uint32) < key
            cnt = cnt + jnp.where(cond, one, 0)
        vpc = DCHUNK // 16
        destbuf[t // vpc, pl.ds((t % vpc) * 16, 16)] = row0 + (cki >> 12) + cnt
        return _

    lax.fori_loop(0, CHUNK // 16, pass_d, None)

    scats = {}
    for d in range(NDMA):
        bi = d % NBUF
        if d >= NBUF:
            scats[d - NBUF].wait()
            g = base + d * DCHUNK
            loads[d] = pltpu.async_copy(
                x_hbm.at[b, pl.ds(g, DCHUNK)], xbuf.at[bi], isems[bi])
        loads[d].wait()
        scats[d] = pltpu.async_copy(
            xbuf.at[bi], out_hbm.at[destbuf.at[d]], osems[bi])

    for d in range(NDMA - NBUF, NDMA):
        scats[d].wait()


_mesh = plsc.VectorSubcoreMesh(core_axis_name="c", subcore_axis_name="s")

_sc_permute = functools.partial(
    pl.kernel,
    out_type=jax.ShapeDtypeStruct((B * L, D), jnp.float32),
    mesh=_mesh,
    compiler_params=pltpu.CompilerParams(needs_layout_passes=False),
    scratch_types=[
        pltpu.VMEM((ROWBUF,), jnp.int32),
        pltpu.VMEM((len(_LUT_PACKED),), jnp.int32),
        pltpu.VMEM((EXTN,), jnp.int32),
        pltpu.VMEM((EXTN,), jnp.int32),
        pltpu.VMEM((EXTN,), jnp.int32),
        pltpu.VMEM((NDMA, DCHUNK), jnp.int32),
        pltpu.VMEM((NBUF, DCHUNK, D), jnp.float32),
        pltpu.SemaphoreType.DMA,
        pltpu.SemaphoreType.DMA,
        pltpu.SemaphoreType.DMA,
        pltpu.SemaphoreType.DMA,
        pltpu.SemaphoreType.DMA,
        pltpu.SemaphoreType.DMA,
    ],
)(_body)


@jax.jit
def kernel(x, atom_type, aa_type):
    comb = (aa_type.astype(jnp.int32) * 64
            + atom_type.astype(jnp.int32)).reshape(B * L)
    out = _sc_permute(x, comb)
    return out.reshape(B, L, D)
